# baseline (speedup 1.0000x reference)
import jax
import jax.numpy as jnp
from jax.experimental import pallas as pl
from jax.experimental.pallas import tpu as pltpu

_NBINS = 51
_BW = 2.0 / (_NBINS - 1)
_INV_BW = 1.0 / _BW
_BS = 1024
_D = 128
_T = 256
_NT = _BS // _T
_NPAIRS = _NT * (_NT + 1) // 2
_NALL = _BS * (_BS - 1) / 2
_T0_ROW = 0
_CNT_ROW = 54
_ACC_ROWS = 56


def _decode(t):
    rb = (t >= 4).astype(jnp.int32) + (t >= 7) + (t >= 9)
    cb = t - (4 * rb - (rb * (rb - 1)) // 2) + rb
    return rb, cb


def _tree8(a):
    return jnp.sum(a.reshape(32, 8, _T), axis=0)


def _sweep(sp, evm, vm, rp_ref, ra_ref):
    for t in range(1, _NBINS):
        r = jnp.maximum(sp - float(t), 0.0)
        ra = r if vm is None else r * vm
        rp = r * evm
        ra_ref[t] = ra_ref[t] + _tree8(ra)
        rp_ref[t] = rp_ref[t] + _tree8(rp)
    t0a = sp if vm is None else sp * vm
    ra_ref[_T0_ROW] = ra_ref[_T0_ROW] + _tree8(t0a)
    rp_ref[_T0_ROW] = rp_ref[_T0_ROW] + _tree8(sp * evm)
    rp_ref[_CNT_ROW] = rp_ref[_CNT_ROW] + _tree8(evm)


def _hist_body(xr_ref, xc_ref, lr_ref, lc_ref, rp_ref, ra_ref):
    t = pl.program_id(0)
    rb, cb = _decode(t)

    @pl.when(t == 0)
    def _init():
        rp_ref[...] = jnp.zeros_like(rp_ref)
        ra_ref[...] = jnp.zeros_like(ra_ref)

    dn = (((1,), (1,)), ((), ()))
    s = jax.lax.dot_general(xr_ref[...], xc_ref[...], dn,
                            preferred_element_type=jnp.float32)
    sp = s * _INV_BW + _INV_BW
    eq = lr_ref[...] == lc_ref[0]

    @pl.when(rb == cb)
    def _diag():
        ii = jax.lax.broadcasted_iota(jnp.int32, (_T, _T), 0)
        jj = jax.lax.broadcasted_iota(jnp.int32, (_T, _T), 1)
        vm = jnp.where(ii == jj, 0.0, 0.5)
        evm = jnp.where(eq, vm, 0.0)
        _sweep(sp, evm, vm, rp_ref, ra_ref)

    @pl.when(rb < cb)
    def _offdiag():
        evm = jnp.where(eq, 1.0, 0.0)
        _sweep(sp, evm, None, rp_ref, ra_ref)


def _finalize_body(rp_ref, ra_ref, out_ref):
    su = jax.lax.broadcasted_iota(jnp.int32, (8, _T), 0)
    jm = jax.lax.broadcasted_iota(jnp.int32, (8, _T), 1) % 8
    cnt_a = 256.0 - 2.0 * jnp.where(su == jm, 1.0, 0.0)

    def hist_rows(ref, cnt):
        rows = [cnt - ref[_T0_ROW] + ref[1],
                ref[_T0_ROW] - 2.0 * ref[1] + ref[2]]
        for k in range(2, _NBINS - 1):
            rows.append(ref[k - 1] - 2.0 * ref[k] + ref[k + 1])
        rows.append(ref[_NBINS - 2] - 2.0 * ref[_NBINS - 1])
        return jnp.concatenate([r.reshape(1, 8, _T) for r in rows], axis=0)

    hp3 = hist_rows(rp_ref, rp_ref[_CNT_ROW])
    ha3 = hist_rows(ra_ref, cnt_a)
    cnt3 = rp_ref[_CNT_ROW].reshape(1, 8, _T)
    hp2 = jnp.sum(jnp.concatenate([hp3, cnt3], axis=0), axis=1)
    ha2 = jnp.sum(ha3, axis=1)
    ones = jnp.ones((1, _T), jnp.float32)
    dn = (((1,), (1,)), ((), ()))
    hpc = jax.lax.dot_general(ones, hp2, dn,
                              preferred_element_type=jnp.float32)
    ha = jax.lax.dot_general(ones, ha2, dn,
                             preferred_element_type=jnp.float32)

    npos = hpc[0:1, _NBINS:_NBINS + 1]
    nneg = _NALL - npos
    hp = hpc[0:1, 0:_NBINS]
    hist_pos = hp / npos
    hist_neg = (ha - hp) / nneg

    m_i = jax.lax.broadcasted_iota(jnp.int32, (_NBINS, _NBINS), 0)
    k_i = jax.lax.broadcasted_iota(jnp.int32, (_NBINS, _NBINS), 1)
    tri = (m_i <= k_i).astype(jnp.float32)
    cdf = jnp.dot(hist_pos, tri, preferred_element_type=jnp.float32)

    out_ref[...] = jnp.sum(hist_neg * cdf, axis=1, keepdims=True)


def kernel(x, labels):
    lab = labels.astype(jnp.int32)
    lab_row = lab.reshape(_BS, 1)
    lab_col = lab.reshape(_NT, 1, _T)

    acc_shape = (_ACC_ROWS, 8, _T)
    rp_acc, ra_acc = pl.pallas_call(
        _hist_body,
        grid=(_NPAIRS,),
        in_specs=[
            pl.BlockSpec((_T, _D), lambda t: (_decode(t)[0], 0)),
            pl.BlockSpec((_T, _D), lambda t: (_decode(t)[1], 0)),
            pl.BlockSpec((_T, 1), lambda t: (_decode(t)[0], 0)),
            pl.BlockSpec((1, 1, _T), lambda t: (_decode(t)[1], 0, 0)),
        ],
        out_specs=[
            pl.BlockSpec(acc_shape, lambda t: (0, 0, 0)),
            pl.BlockSpec(acc_shape, lambda t: (0, 0, 0)),
        ],
        out_shape=[
            jax.ShapeDtypeStruct(acc_shape, jnp.float32),
            jax.ShapeDtypeStruct(acc_shape, jnp.float32),
        ],
        compiler_params=pltpu.CompilerParams(
            dimension_semantics=("arbitrary",)),
    )(x, x, lab_row, lab_col)

    loss = pl.pallas_call(
        _finalize_body,
        out_shape=jax.ShapeDtypeStruct((1, 1), jnp.float32),
    )(rp_acc, ra_acc)
    return loss[0, 0]


# speedup vs baseline: 1.0877x; 1.0877x over previous
import jax
import jax.numpy as jnp
from jax.experimental import pallas as pl
from jax.experimental.pallas import tpu as pltpu

_NBINS = 51
_BW = 2.0 / (_NBINS - 1)
_INV_BW = 1.0 / _BW
_BS = 1024
_D = 128
_T = 128
_NT = _BS // _T
_NPAIRS = _NT * (_NT + 1) // 2
_NALL = _BS * (_BS - 1) / 2
_T0_ROW = 0
_CNT_ROW = 54
_ACC_ROWS = 56


def _decode(t):
    rb = (
        (t >= 8).astype(jnp.int32) + (t >= 15) + (t >= 21) + (t >= 26)
        + (t >= 30) + (t >= 33) + (t >= 35)
    )
    cb = t - (8 * rb - (rb * (rb - 1)) // 2) + rb
    return rb, cb


def _tree8(a):
    return jnp.sum(a.reshape(16, 8, _T), axis=0)


def _sweep(sp, evm, vm, rp_ref, ra_ref):
    for t in range(1, _NBINS):
        r = jnp.maximum(sp - float(t), 0.0)
        ra = r if vm is None else r * vm
        rp = r * evm
        ra_ref[t] = ra_ref[t] + _tree8(ra)
        rp_ref[t] = rp_ref[t] + _tree8(rp)
    t0a = sp if vm is None else sp * vm
    ra_ref[_T0_ROW] = ra_ref[_T0_ROW] + _tree8(t0a)
    rp_ref[_T0_ROW] = rp_ref[_T0_ROW] + _tree8(sp * evm)
    rp_ref[_CNT_ROW] = rp_ref[_CNT_ROW] + _tree8(evm)


def _hist_body(xr_ref, xc_ref, lr_ref, lc_ref, rp_ref, ra_ref):
    t = pl.program_id(0)
    rb, cb = _decode(t)

    @pl.when(t == 0)
    def _init():
        rp_ref[...] = jnp.zeros_like(rp_ref)
        ra_ref[...] = jnp.zeros_like(ra_ref)

    dn = (((1,), (1,)), ((), ()))
    s = jax.lax.dot_general(xr_ref[...], xc_ref[...], dn,
                            preferred_element_type=jnp.float32)
    sp = s * _INV_BW + _INV_BW
    eq = lr_ref[...] == lc_ref[0]

    @pl.when(rb == cb)
    def _diag():
        ii = jax.lax.broadcasted_iota(jnp.int32, (_T, _T), 0)
        jj = jax.lax.broadcasted_iota(jnp.int32, (_T, _T), 1)
        vm = jnp.where(ii == jj, 0.0, 0.5)
        evm = jnp.where(eq, vm, 0.0)
        _sweep(sp, evm, vm, rp_ref, ra_ref)

    @pl.when(rb < cb)
    def _offdiag():
        evm = jnp.where(eq, 1.0, 0.0)
        _sweep(sp, evm, None, rp_ref, ra_ref)


def _finalize_body(rp_ref, ra_ref, out_ref):
    su = jax.lax.broadcasted_iota(jnp.int32, (8, _T), 0)
    jm = jax.lax.broadcasted_iota(jnp.int32, (8, _T), 1) % 8
    cnt_a = 512.0 - 4.0 * jnp.where(su == jm, 1.0, 0.0)

    def hist_rows(ref, cnt):
        rows = [cnt - ref[_T0_ROW] + ref[1],
                ref[_T0_ROW] - 2.0 * ref[1] + ref[2]]
        for k in range(2, _NBINS - 1):
            rows.append(ref[k - 1] - 2.0 * ref[k] + ref[k + 1])
        rows.append(ref[_NBINS - 2] - 2.0 * ref[_NBINS - 1])
        return jnp.concatenate([r.reshape(1, 8, _T) for r in rows], axis=0)

    hp3 = hist_rows(rp_ref, rp_ref[_CNT_ROW])
    ha3 = hist_rows(ra_ref, cnt_a)
    cnt3 = rp_ref[_CNT_ROW].reshape(1, 8, _T)
    hp2 = jnp.sum(jnp.concatenate([hp3, cnt3], axis=0), axis=1)
    ha2 = jnp.sum(ha3, axis=1)
    ones = jnp.ones((1, _T), jnp.float32)
    dn = (((1,), (1,)), ((), ()))
    hpc = jax.lax.dot_general(ones, hp2, dn,
                              preferred_element_type=jnp.float32)
    ha = jax.lax.dot_general(ones, ha2, dn,
                             preferred_element_type=jnp.float32)

    npos = hpc[0:1, _NBINS:_NBINS + 1]
    nneg = _NALL - npos
    hp = hpc[0:1, 0:_NBINS]
    hist_pos = hp / npos
    hist_neg = (ha - hp) / nneg

    m_i = jax.lax.broadcasted_iota(jnp.int32, (_NBINS, _NBINS), 0)
    k_i = jax.lax.broadcasted_iota(jnp.int32, (_NBINS, _NBINS), 1)
    tri = (m_i <= k_i).astype(jnp.float32)
    cdf = jnp.dot(hist_pos, tri, preferred_element_type=jnp.float32)

    out_ref[...] = jnp.sum(hist_neg * cdf, axis=1, keepdims=True)


def kernel(x, labels):
    lab = labels.astype(jnp.int32)
    lab_row = lab.reshape(_BS, 1)
    lab_col = lab.reshape(_NT, 1, _T)

    acc_shape = (_ACC_ROWS, 8, _T)
    rp_acc, ra_acc = pl.pallas_call(
        _hist_body,
        grid=(_NPAIRS,),
        in_specs=[
            pl.BlockSpec((_T, _D), lambda t: (_decode(t)[0], 0)),
            pl.BlockSpec((_T, _D), lambda t: (_decode(t)[1], 0)),
            pl.BlockSpec((_T, 1), lambda t: (_decode(t)[0], 0)),
            pl.BlockSpec((1, 1, _T), lambda t: (_decode(t)[1], 0, 0)),
        ],
        out_specs=[
            pl.BlockSpec(acc_shape, lambda t: (0, 0, 0)),
            pl.BlockSpec(acc_shape, lambda t: (0, 0, 0)),
        ],
        out_shape=[
            jax.ShapeDtypeStruct(acc_shape, jnp.float32),
            jax.ShapeDtypeStruct(acc_shape, jnp.float32),
        ],
        compiler_params=pltpu.CompilerParams(
            dimension_semantics=("arbitrary",)),
    )(x, x, lab_row, lab_col)

    loss = pl.pallas_call(
        _finalize_body,
        out_shape=jax.ShapeDtypeStruct((1, 1), jnp.float32),
    )(rp_acc, ra_acc)
    return loss[0, 0]


# speedup vs baseline: 1.1558x; 1.0626x over previous
import jax
import jax.numpy as jnp
from jax.experimental import pallas as pl
from jax.experimental.pallas import tpu as pltpu

_NBINS = 51
_BW = 2.0 / (_NBINS - 1)
_INV_BW = 1.0 / _BW
_BS = 1024
_D = 128
_T = 128
_NT = _BS // _T
_NALL = _BS * (_BS - 1) / 2
_T0_ROW = 0
_CNT_ROW = 54
_ACC_ROWS = 56


def _tree8(a):
    return jnp.sum(a.reshape(16, 8, _T), axis=0)


def _sweep(sp, evm, vm, rp_ref, ra_ref):
    for t in range(1, _NBINS):
        r = jnp.maximum(sp - float(t), 0.0)
        ra = r if vm is None else r * vm
        rp = r * evm
        ra_ref[t] = ra_ref[t] + _tree8(ra)
        rp_ref[t] = rp_ref[t] + _tree8(rp)
    t0a = sp if vm is None else sp * vm
    ra_ref[_T0_ROW] = ra_ref[_T0_ROW] + _tree8(t0a)
    rp_ref[_T0_ROW] = rp_ref[_T0_ROW] + _tree8(sp * evm)
    rp_ref[_CNT_ROW] = rp_ref[_CNT_ROW] + _tree8(evm)


def _body(x_ref, lr_ref, lc_ref, out_ref, rp_ref, ra_ref):
    rp_ref[...] = jnp.zeros_like(rp_ref)
    ra_ref[...] = jnp.zeros_like(ra_ref)

    ii = jax.lax.broadcasted_iota(jnp.int32, (_T, _T), 0)
    jj = jax.lax.broadcasted_iota(jnp.int32, (_T, _T), 1)
    diag_vm = jnp.where(ii == jj, 0.0, 0.5)
    dn = (((1,), (1,)), ((), ()))

    for rb in range(_NT):
        xr = x_ref[rb * _T:(rb + 1) * _T, :]
        lr = lr_ref[rb * _T:(rb + 1) * _T, :]
        for cb in range(rb, _NT):
            xc = x_ref[cb * _T:(cb + 1) * _T, :]
            s = jax.lax.dot_general(xr, xc, dn,
                                    preferred_element_type=jnp.float32)
            sp = s * _INV_BW + _INV_BW
            eq = lr == lc_ref[cb]
            if rb == cb:
                evm = jnp.where(eq, diag_vm, 0.0)
                _sweep(sp, evm, diag_vm, rp_ref, ra_ref)
            else:
                evm = jnp.where(eq, 1.0, 0.0)
                _sweep(sp, evm, None, rp_ref, ra_ref)

    su = jax.lax.broadcasted_iota(jnp.int32, (8, _T), 0)
    jm = jax.lax.broadcasted_iota(jnp.int32, (8, _T), 1) % 8
    cnt_a = 512.0 - 4.0 * jnp.where(su == jm, 1.0, 0.0)

    def hist_rows(ref, cnt):
        rows = [cnt - ref[_T0_ROW] + ref[1],
                ref[_T0_ROW] - 2.0 * ref[1] + ref[2]]
        for k in range(2, _NBINS - 1):
            rows.append(ref[k - 1] - 2.0 * ref[k] + ref[k + 1])
        rows.append(ref[_NBINS - 2] - 2.0 * ref[_NBINS - 1])
        return jnp.concatenate([r.reshape(1, 8, _T) for r in rows], axis=0)

    hp3 = hist_rows(rp_ref, rp_ref[_CNT_ROW])
    ha3 = hist_rows(ra_ref, cnt_a)
    cnt3 = rp_ref[_CNT_ROW].reshape(1, 8, _T)
    hp2 = jnp.sum(jnp.concatenate([hp3, cnt3], axis=0), axis=1)
    ha2 = jnp.sum(ha3, axis=1)
    ones = jnp.ones((1, _T), jnp.float32)
    hpc = jax.lax.dot_general(ones, hp2, (((1,), (1,)), ((), ())),
                              preferred_element_type=jnp.float32)
    ha = jax.lax.dot_general(ones, ha2, (((1,), (1,)), ((), ())),
                             preferred_element_type=jnp.float32)

    npos = hpc[0:1, _NBINS:_NBINS + 1]
    nneg = _NALL - npos
    hp = hpc[0:1, 0:_NBINS]
    hist_pos = hp / npos
    hist_neg = (ha - hp) / nneg

    m_i = jax.lax.broadcasted_iota(jnp.int32, (_NBINS, _NBINS), 0)
    k_i = jax.lax.broadcasted_iota(jnp.int32, (_NBINS, _NBINS), 1)
    tri = (m_i <= k_i).astype(jnp.float32)
    cdf = jnp.dot(hist_pos, tri, preferred_element_type=jnp.float32)

    out_ref[...] = jnp.sum(hist_neg * cdf, axis=1, keepdims=True)


def kernel(x, labels):
    lab = labels.astype(jnp.int32)
    lab_row = lab.reshape(_BS, 1)
    lab_col = lab.reshape(_NT, 1, _T)

    loss = pl.pallas_call(
        _body,
        scratch_shapes=[
            pltpu.VMEM((_ACC_ROWS, 8, _T), jnp.float32),
            pltpu.VMEM((_ACC_ROWS, 8, _T), jnp.float32),
        ],
        out_shape=jax.ShapeDtypeStruct((1, 1), jnp.float32),
    )(x, lab_row, lab_col)
    return loss[0, 0]
